# baseline (speedup 1.0000x reference)
import jax
import jax.numpy as jnp
from jax.experimental import pallas as pl
from jax.experimental.pallas import tpu as pltpu

N = 10000
F = 128
BM = 200
NB = N // BM


def _hamilton(w):
    r, i, j, k = jnp.split(w, 4, axis=1)
    r2 = jnp.concatenate([r, -i, -j, -k], axis=0)
    i2 = jnp.concatenate([i, r, -k, j], axis=0)
    j2 = jnp.concatenate([j, k, r, -i], axis=0)
    k2 = jnp.concatenate([k, -j, i, r], axis=0)
    return jnp.concatenate([r2, i2, j2, k2], axis=1)


def _body(x_ref, w1_ref, w2_ref, g_ref, b_ref, adj_ref, out_ref, t_s, raw_s, stats_s):
    p = pl.program_id(0)
    i = pl.program_id(1)

    @pl.when((p == 0) & (i == 0))
    def _front():
        h1 = _hamilton(w1_ref[...])
        h2 = _hamilton(w2_ref[...])
        o1 = jnp.tanh(jnp.dot(x_ref[...], h1, preferred_element_type=jnp.float32))
        t_s[...] = jnp.dot(o1, h2, preferred_element_type=jnp.float32).astype(jnp.bfloat16)
        stats_s[...] = jnp.zeros_like(stats_s)

    @pl.when(p == 0)
    def _spmm():
        acc = jnp.dot(
            adj_ref[...].astype(jnp.bfloat16), t_s[...],
            preferred_element_type=jnp.float32,
        )
        raw_s[pl.ds(i * BM, BM), :] = acc
        stats_s[0:1, :] += jnp.sum(acc, axis=0, keepdims=True)
        stats_s[1:2, :] += jnp.sum(acc * acc, axis=0, keepdims=True)

    @pl.when(p == 1)
    def _bn():
        inv_n = 1.0 / N
        mean = stats_s[0:1, :] * inv_n
        var = stats_s[1:2, :] * inv_n - mean * mean
        scale = jax.lax.rsqrt(var + 1e-5) * g_ref[...]
        out_ref[...] = (raw_s[pl.ds(i * BM, BM), :] - mean) * scale + b_ref[...]


def kernel(input, adj, weight1, weight2, gamma, beta):
    return pl.pallas_call(
        _body,
        grid=(2, NB),
        in_specs=[
            pl.BlockSpec((N, F), lambda p, i: (0, 0)),
            pl.BlockSpec((F // 4, F), lambda p, i: (0, 0)),
            pl.BlockSpec((F // 4, F), lambda p, i: (0, 0)),
            pl.BlockSpec((1, F), lambda p, i: (0, 0)),
            pl.BlockSpec((1, F), lambda p, i: (0, 0)),
            pl.BlockSpec((BM, N), lambda p, i: (jnp.where(p == 0, i, NB - 1), 0)),
        ],
        out_specs=pl.BlockSpec((BM, F), lambda p, i: (jnp.where(p == 0, 0, i), 0)),
        out_shape=jax.ShapeDtypeStruct((N, F), jnp.float32),
        scratch_shapes=[
            pltpu.VMEM((N, F), jnp.bfloat16),
            pltpu.VMEM((N, F), jnp.float32),
            pltpu.VMEM((8, F), jnp.float32),
        ],
        compiler_params=pltpu.CompilerParams(
            dimension_semantics=("arbitrary", "arbitrary"),
        ),
    )(input, weight1, weight2, gamma.reshape(1, F), beta.reshape(1, F), adj)


# speedup vs baseline: 1.0709x; 1.0709x over previous
import jax
import jax.numpy as jnp
from jax.experimental import pallas as pl
from jax.experimental.pallas import tpu as pltpu

N = 10000
F = 128
BM = 400
NB = N // BM


def _hamilton(w):
    r, i, j, k = jnp.split(w, 4, axis=1)
    r2 = jnp.concatenate([r, -i, -j, -k], axis=0)
    i2 = jnp.concatenate([i, r, -k, j], axis=0)
    j2 = jnp.concatenate([j, k, r, -i], axis=0)
    k2 = jnp.concatenate([k, -j, i, r], axis=0)
    return jnp.concatenate([r2, i2, j2, k2], axis=1)


def _body(x_ref, w1_ref, w2_ref, g_ref, b_ref, adj_ref, out_ref, t_s, raw_s, stats_s):
    p = pl.program_id(0)
    i = pl.program_id(1)

    @pl.when((p == 0) & (i == 0))
    def _front():
        h1 = _hamilton(w1_ref[...])
        h2 = _hamilton(w2_ref[...])
        o1 = jnp.tanh(jnp.dot(x_ref[...], h1, preferred_element_type=jnp.float32))
        t_s[...] = jnp.dot(o1, h2, preferred_element_type=jnp.float32).astype(jnp.bfloat16)
        stats_s[...] = jnp.zeros_like(stats_s)

    @pl.when(p == 0)
    def _spmm():
        acc = jnp.dot(
            adj_ref[...].astype(jnp.bfloat16), t_s[...],
            preferred_element_type=jnp.float32,
        )
        raw_s[pl.ds(i * BM, BM), :] = acc
        stats_s[0:1, :] += jnp.sum(acc, axis=0, keepdims=True)
        stats_s[1:2, :] += jnp.sum(acc * acc, axis=0, keepdims=True)

    @pl.when(p == 1)
    def _bn():
        inv_n = 1.0 / N
        mean = stats_s[0:1, :] * inv_n
        var = stats_s[1:2, :] * inv_n - mean * mean
        scale = jax.lax.rsqrt(var + 1e-5) * g_ref[...]
        out_ref[...] = (raw_s[pl.ds(i * BM, BM), :] - mean) * scale + b_ref[...]


def kernel(input, adj, weight1, weight2, gamma, beta):
    return pl.pallas_call(
        _body,
        grid=(2, NB),
        in_specs=[
            pl.BlockSpec((N, F), lambda p, i: (0, 0)),
            pl.BlockSpec((F // 4, F), lambda p, i: (0, 0)),
            pl.BlockSpec((F // 4, F), lambda p, i: (0, 0)),
            pl.BlockSpec((1, F), lambda p, i: (0, 0)),
            pl.BlockSpec((1, F), lambda p, i: (0, 0)),
            pl.BlockSpec((BM, N), lambda p, i: (jnp.where(p == 0, i, NB - 1), 0)),
        ],
        out_specs=pl.BlockSpec((BM, F), lambda p, i: (jnp.where(p == 0, 0, i), 0)),
        out_shape=jax.ShapeDtypeStruct((N, F), jnp.float32),
        scratch_shapes=[
            pltpu.VMEM((N, F), jnp.bfloat16),
            pltpu.VMEM((N, F), jnp.float32),
            pltpu.VMEM((8, F), jnp.float32),
        ],
        compiler_params=pltpu.CompilerParams(
            dimension_semantics=("arbitrary", "arbitrary"),
        ),
    )(input, weight1, weight2, gamma.reshape(1, F), beta.reshape(1, F), adj)


# speedup vs baseline: 1.1125x; 1.0388x over previous
import jax
import jax.numpy as jnp
from jax.experimental import pallas as pl
from jax.experimental.pallas import tpu as pltpu

N = 10000
F = 128
BM = 400
NB = N // BM
BO = 2000
NO = N // BO


def _hamilton(w):
    r, i, j, k = jnp.split(w, 4, axis=1)
    r2 = jnp.concatenate([r, -i, -j, -k], axis=0)
    i2 = jnp.concatenate([i, r, -k, j], axis=0)
    j2 = jnp.concatenate([j, k, r, -i], axis=0)
    k2 = jnp.concatenate([k, -j, i, r], axis=0)
    return jnp.concatenate([r2, i2, j2, k2], axis=1)


def _body(x_ref, w1_ref, w2_ref, g_ref, b_ref, adj_ref, out_ref, t_s, raw_s, stats_s):
    s = pl.program_id(0)

    @pl.when(s == 0)
    def _front():
        h1 = _hamilton(w1_ref[...]).astype(jnp.bfloat16)
        h2 = _hamilton(w2_ref[...]).astype(jnp.bfloat16)
        x16 = x_ref[...].astype(jnp.bfloat16)
        o1 = jnp.tanh(jnp.dot(x16, h1, preferred_element_type=jnp.float32))
        t_s[...] = jnp.dot(
            o1.astype(jnp.bfloat16), h2, preferred_element_type=jnp.float32
        ).astype(jnp.bfloat16)
        stats_s[...] = jnp.zeros_like(stats_s)

    @pl.when(s < NB)
    def _spmm():
        acc = jnp.dot(
            adj_ref[...].astype(jnp.bfloat16), t_s[...],
            preferred_element_type=jnp.float32,
        )
        raw_s[pl.ds(s * BM, BM), :] = acc
        stats_s[0:1, :] += jnp.sum(acc, axis=0, keepdims=True)
        stats_s[1:2, :] += jnp.sum(acc * acc, axis=0, keepdims=True)

    @pl.when(s >= NB)
    def _bn():
        inv_n = 1.0 / N
        mean = stats_s[0:1, :] * inv_n
        var = stats_s[1:2, :] * inv_n - mean * mean
        scale = jax.lax.rsqrt(var + 1e-5) * g_ref[...]
        i = s - NB
        out_ref[...] = (raw_s[pl.ds(i * BO, BO), :] - mean) * scale + b_ref[...]


def kernel(input, adj, weight1, weight2, gamma, beta):
    return pl.pallas_call(
        _body,
        grid=(NB + NO,),
        in_specs=[
            pl.BlockSpec((N, F), lambda s: (0, 0)),
            pl.BlockSpec((F // 4, F), lambda s: (0, 0)),
            pl.BlockSpec((F // 4, F), lambda s: (0, 0)),
            pl.BlockSpec((1, F), lambda s: (0, 0)),
            pl.BlockSpec((1, F), lambda s: (0, 0)),
            pl.BlockSpec((BM, N), lambda s: (jnp.where(s < NB, s, NB - 1), 0)),
        ],
        out_specs=pl.BlockSpec((BO, F), lambda s: (jnp.where(s < NB, 0, s - NB), 0)),
        out_shape=jax.ShapeDtypeStruct((N, F), jnp.float32),
        scratch_shapes=[
            pltpu.VMEM((N, F), jnp.bfloat16),
            pltpu.VMEM((N, F), jnp.float32),
            pltpu.VMEM((8, F), jnp.float32),
        ],
        compiler_params=pltpu.CompilerParams(
            dimension_semantics=("arbitrary",),
        ),
    )(input, weight1, weight2, gamma.reshape(1, F), beta.reshape(1, F), adj)


# speedup vs baseline: 1.1211x; 1.0077x over previous
import jax
import jax.numpy as jnp
from jax.experimental import pallas as pl
from jax.experimental.pallas import tpu as pltpu

N = 10000
F = 128
BM = 400
NB = N // BM
BO = 10000
NO = N // BO


def _hamilton(w):
    r, i, j, k = jnp.split(w, 4, axis=1)
    r2 = jnp.concatenate([r, -i, -j, -k], axis=0)
    i2 = jnp.concatenate([i, r, -k, j], axis=0)
    j2 = jnp.concatenate([j, k, r, -i], axis=0)
    k2 = jnp.concatenate([k, -j, i, r], axis=0)
    return jnp.concatenate([r2, i2, j2, k2], axis=1)


def _body(x_ref, w1_ref, w2_ref, g_ref, b_ref, adj_ref, out_ref, t_s, raw_s, stats_s):
    s = pl.program_id(0)

    @pl.when(s == 0)
    def _front():
        h1 = _hamilton(w1_ref[...]).astype(jnp.bfloat16)
        h2 = _hamilton(w2_ref[...]).astype(jnp.bfloat16)
        x16 = x_ref[...].astype(jnp.bfloat16)
        o1 = jnp.tanh(jnp.dot(x16, h1, preferred_element_type=jnp.float32))
        t_s[...] = jnp.dot(
            o1.astype(jnp.bfloat16), h2, preferred_element_type=jnp.float32
        ).astype(jnp.bfloat16)
        stats_s[...] = jnp.zeros_like(stats_s)

    @pl.when(s < NB)
    def _spmm():
        acc = jnp.dot(
            adj_ref[...].astype(jnp.bfloat16), t_s[...],
            preferred_element_type=jnp.float32,
        )
        raw_s[pl.ds(s * BM, BM), :] = acc
        stats_s[0:1, :] += jnp.sum(acc, axis=0, keepdims=True)
        stats_s[1:2, :] += jnp.sum(acc * acc, axis=0, keepdims=True)

    @pl.when(s >= NB)
    def _bn():
        inv_n = 1.0 / N
        mean = stats_s[0:1, :] * inv_n
        var = stats_s[1:2, :] * inv_n - mean * mean
        scale = jax.lax.rsqrt(var + 1e-5) * g_ref[...]
        i = s - NB
        out_ref[...] = (raw_s[pl.ds(i * BO, BO), :] - mean) * scale + b_ref[...]


def kernel(input, adj, weight1, weight2, gamma, beta):
    return pl.pallas_call(
        _body,
        grid=(NB + NO,),
        in_specs=[
            pl.BlockSpec((N, F), lambda s: (0, 0)),
            pl.BlockSpec((F // 4, F), lambda s: (0, 0)),
            pl.BlockSpec((F // 4, F), lambda s: (0, 0)),
            pl.BlockSpec((1, F), lambda s: (0, 0)),
            pl.BlockSpec((1, F), lambda s: (0, 0)),
            pl.BlockSpec((BM, N), lambda s: (jnp.where(s < NB, s, NB - 1), 0)),
        ],
        out_specs=pl.BlockSpec((BO, F), lambda s: (jnp.where(s < NB, 0, s - NB), 0)),
        out_shape=jax.ShapeDtypeStruct((N, F), jnp.float32),
        scratch_shapes=[
            pltpu.VMEM((N, F), jnp.bfloat16),
            pltpu.VMEM((N, F), jnp.float32),
            pltpu.VMEM((8, F), jnp.float32),
        ],
        compiler_params=pltpu.CompilerParams(
            dimension_semantics=("arbitrary",),
        ),
    )(input, weight1, weight2, gamma.reshape(1, F), beta.reshape(1, F), adj)
